# baseline (speedup 1.0000x reference)
import functools

import jax
import jax.numpy as jnp
from jax import lax
from jax.experimental import pallas as pl
from jax.experimental.pallas import tpu as pltpu
from jax.experimental.pallas import tpu_sc as plsc

_GRID = (128, 128, 128)
_N = 50000
_NW = 32
_RPW = 1600
_NP = _NW * _RPW
_C = 128
_K = 9
_B = 64
_CHUNKS = _RPW // _B
_BM = 512

_OFF_A = tuple((0, dy, dx) for dy in (-1, 0, 1) for dx in (-1, 0, 1))
_OFF_B = tuple((dz, 0, dx) for dz in (-1, 0, 1) for dx in (-1, 0, 1))


def _nbr_idx(grid, coords, offsets):
    D, H, Wd = _GRID
    z, y, x = coords[:, 0], coords[:, 1], coords[:, 2]
    cols = []
    for k, (oz, oy, ox) in enumerate(offsets):
        nz, ny, nx = z + oz, y + oy, x + ox
        valid = (nz >= 0) & (nz < D) & (ny >= 0) & (ny < H) & (nx >= 0) & (nx < Wd)
        nb = grid[jnp.clip(nz, 0, D - 1), jnp.clip(ny, 0, H - 1), jnp.clip(nx, 0, Wd - 1)]
        nb = jnp.where(valid & (nb >= 0), nb, _N)
        cols.append(nb * _K + k)
    idx = jnp.stack(cols)
    pad = jnp.broadcast_to((_N * _K + jnp.arange(_K, dtype=jnp.int32))[:, None],
                           (_K, _NP - _N))
    return jnp.concatenate([idx, pad], axis=1)


def _cat9(W):
    return W.reshape(_K, _C, _C).transpose(1, 0, 2).reshape(_C, _K * _C)


def _mm_plain_body(x_ref, w_ref, y_ref):
    y_ref[...] = jnp.dot(x_ref[...], w_ref[...], preferred_element_type=jnp.float32)


def _mm_plain(x, wcat):
    n_out = wcat.shape[1]
    return pl.pallas_call(
        _mm_plain_body,
        grid=(_NP // _BM,),
        in_specs=[pl.BlockSpec((_BM, _C), lambda i: (i, 0)),
                  pl.BlockSpec((_C, n_out), lambda i: (0, 0))],
        out_specs=pl.BlockSpec((_BM, n_out), lambda i: (i, 0)),
        out_shape=jax.ShapeDtypeStruct((_NP, n_out), jnp.float32),
    )(x, wcat)


def _bn_affine(p, bnw, bnb):
    psum = jnp.sum(p, axis=0)
    mu = psum[0:1] / float(_N)
    msq = psum[1:2] / float(_N)
    var = msq - mu * mu
    s = bnw * lax.rsqrt(var + 1e-5)
    t = bnb - mu * s
    return s, t


def _mm_fused_body(l_ref, w_ref, bnw_ref, bnb_ref, p_ref, y_ref):
    s, t = _bn_affine(p_ref[...], bnw_ref[...], bnb_ref[...])
    rows = pl.program_id(0) * _BM + lax.broadcasted_iota(jnp.int32, (_BM, 1), 0)
    a = jnp.where(rows < _N, l_ref[...] * s + t, 0.0)
    y_ref[...] = jnp.dot(a, w_ref[...], preferred_element_type=jnp.float32)


def _mm_fused(lmat, wcat, bnw, bnb, p):
    n_out = wcat.shape[1]
    return pl.pallas_call(
        _mm_fused_body,
        grid=(_NP // _BM,),
        in_specs=[pl.BlockSpec((_BM, _C), lambda i: (i, 0)),
                  pl.BlockSpec((_C, n_out), lambda i: (0, 0)),
                  pl.BlockSpec((1, _C), lambda i: (0, 0)),
                  pl.BlockSpec((1, _C), lambda i: (0, 0)),
                  pl.BlockSpec((_NW, 2, _C), lambda i: (0, 0, 0))],
        out_specs=pl.BlockSpec((_BM, n_out), lambda i: (i, 0)),
        out_shape=jax.ShapeDtypeStruct((_NP, n_out), jnp.float32),
    )(lmat, wcat, bnw, bnb, p)


def _final_body(l2_ref, l4_ref, p2_ref, p4_ref, w2_ref, b2_ref, w4_ref, b4_ref, o_ref):
    s2, t2 = _bn_affine(p2_ref[...], w2_ref[...], b2_ref[...])
    s4, t4 = _bn_affine(p4_ref[...], w4_ref[...], b4_ref[...])
    o_ref[...] = (l2_ref[...] * s2 + t2) + (l4_ref[...] * s4 + t4)


def _final(l2, p2, w2, b2, l4, p4, w4, b4):
    return pl.pallas_call(
        _final_body,
        grid=(_NP // _BM,),
        in_specs=[pl.BlockSpec((_BM, _C), lambda i: (i, 0)),
                  pl.BlockSpec((_BM, _C), lambda i: (i, 0)),
                  pl.BlockSpec((_NW, 2, _C), lambda i: (0, 0, 0)),
                  pl.BlockSpec((_NW, 2, _C), lambda i: (0, 0, 0)),
                  pl.BlockSpec((1, _C), lambda i: (0, 0)),
                  pl.BlockSpec((1, _C), lambda i: (0, 0)),
                  pl.BlockSpec((1, _C), lambda i: (0, 0)),
                  pl.BlockSpec((1, _C), lambda i: (0, 0))],
        out_specs=pl.BlockSpec((_BM, _C), lambda i: (i, 0)),
        out_shape=jax.ShapeDtypeStruct((_NP, _C), jnp.float32),
    )(l2, l4, p2, p4, w2, b2, w4, b4)


def _gather_body(tab_hbm, idx_hbm, l_hbm, p_hbm, idx_v, rows_v, out_v, st_v, sem):
    wid = lax.axis_index("s") * 2 + lax.axis_index("c")
    zero16 = jnp.zeros((16,), jnp.float32)
    for a in range(2):
        for c in range(8):
            st_v[a, pl.ds(c * 16, 16)] = zero16

    def chunk(cc, carry):
        base = wid * _RPW + cc * _B
        for kk in range(_K):
            pltpu.sync_copy(idx_hbm.at[kk, pl.ds(base, _B)], idx_v.at[kk])
        descs = [pltpu.async_copy(tab_hbm.at[idx_v.at[kk]], rows_v.at[kk], sem)
                 for kk in range(_K)]
        for d in descs:
            d.wait()

        def row(r, carry2):
            for c in range(8):
                sl = pl.ds(c * 16, 16)
                v = rows_v[0, r, sl]
                for kk in range(1, _K):
                    v = v + rows_v[kk, r, sl]
                v = jnp.where(v >= 0.0, v, v * 0.01)
                out_v[r, sl] = v
                plsc.addupdate(st_v.at[0, sl], v)
                plsc.addupdate(st_v.at[1, sl], v * v)
            return carry2

        lax.fori_loop(0, _B, row, 0)
        pltpu.sync_copy(out_v, l_hbm.at[pl.ds(base, _B)])
        return carry

    lax.fori_loop(0, _CHUNKS, chunk, 0)
    pltpu.sync_copy(st_v, p_hbm.at[wid])


def _gather_sc(tab, idx):
    mesh = plsc.VectorSubcoreMesh(core_axis_name="c", subcore_axis_name="s")
    f = functools.partial(
        pl.kernel,
        out_type=(jax.ShapeDtypeStruct((_NP, _C), jnp.float32),
                  jax.ShapeDtypeStruct((_NW, 2, _C), jnp.float32)),
        mesh=mesh,
        scratch_types=[pltpu.VMEM((_K, _B), jnp.int32),
                       pltpu.VMEM((_K, _B, _C), jnp.float32),
                       pltpu.VMEM((_B, _C), jnp.float32),
                       pltpu.VMEM((2, _C), jnp.float32),
                       pltpu.SemaphoreType.DMA],
    )(_gather_body)
    return f(tab, idx)


def kernel(features, coords, W1, W1_2, W2, W3,
           bn0_w, bn0_b, bn0_2_w, bn0_2_b, bn1_w, bn1_b, bn2_w, bn2_b):
    f32 = jnp.float32
    x = jnp.zeros((_NP, _C), f32).at[:_N].set(features)
    grid = jnp.full(_GRID, -1, jnp.int32).at[
        coords[:, 0], coords[:, 1], coords[:, 2]].set(jnp.arange(_N, dtype=jnp.int32))
    idx_a = _nbr_idx(grid, coords, _OFF_A)
    idx_b = _nbr_idx(grid, coords, _OFF_B)

    w1c, w12c, w2c, w3c = _cat9(W1), _cat9(W1_2), _cat9(W2), _cat9(W3)
    bn = lambda v: v.reshape(1, _C)

    y1 = _mm_plain(x, w1c).reshape(_NP * _K, _C)
    l1, p1 = _gather_sc(y1, idx_a)
    y2 = _mm_fused(l1, w12c, bn(bn0_w), bn(bn0_b), p1).reshape(_NP * _K, _C)
    l2, p2 = _gather_sc(y2, idx_b)

    y3 = _mm_plain(x, w2c).reshape(_NP * _K, _C)
    l3, p3 = _gather_sc(y3, idx_b)
    y4 = _mm_fused(l3, w3c, bn(bn1_w), bn(bn1_b), p3).reshape(_NP * _K, _C)
    l4, p4 = _gather_sc(y4, idx_a)

    out = _final(l2, p2, bn(bn0_2_w), bn(bn0_2_b), l4, p4, bn(bn2_w), bn(bn2_b))
    return out[:_N]


# speedup vs baseline: 1.0055x; 1.0055x over previous
import functools

import jax
import jax.numpy as jnp
from jax import lax
from jax.experimental import pallas as pl
from jax.experimental.pallas import tpu as pltpu
from jax.experimental.pallas import tpu_sc as plsc

_GRID = (128, 128, 128)
_N = 50000
_NW = 32
_RPW = 1600
_NP = _NW * _RPW
_C = 128
_K = 9
_B = 32
_CHUNKS = _RPW // _B
_BM = 512

_OFF_A = tuple((0, dy, dx) for dy in (-1, 0, 1) for dx in (-1, 0, 1))
_OFF_B = tuple((dz, 0, dx) for dz in (-1, 0, 1) for dx in (-1, 0, 1))


def _nbr_idx(grid, coords, offsets):
    D, H, Wd = _GRID
    z, y, x = coords[:, 0], coords[:, 1], coords[:, 2]
    cols = []
    for k, (oz, oy, ox) in enumerate(offsets):
        nz, ny, nx = z + oz, y + oy, x + ox
        valid = (nz >= 0) & (nz < D) & (ny >= 0) & (ny < H) & (nx >= 0) & (nx < Wd)
        nb = grid[jnp.clip(nz, 0, D - 1), jnp.clip(ny, 0, H - 1), jnp.clip(nx, 0, Wd - 1)]
        nb = jnp.where(valid & (nb >= 0), nb, _N)
        cols.append(nb * _K + k)
    idx = jnp.stack(cols)
    pad = jnp.broadcast_to((_N * _K + jnp.arange(_K, dtype=jnp.int32))[:, None],
                           (_K, _NP - _N))
    idx = jnp.concatenate([idx, pad], axis=1)
    return idx.reshape(_K, _NW, _RPW).transpose(1, 0, 2)


def _cat9(W):
    return W.reshape(_K, _C, _C).transpose(1, 0, 2).reshape(_C, _K * _C)


def _mm_plain_body(x_ref, w_ref, y_ref):
    y_ref[...] = jnp.dot(x_ref[...], w_ref[...], preferred_element_type=jnp.float32)


def _mm_plain(x, wcat):
    n_out = wcat.shape[1]
    return pl.pallas_call(
        _mm_plain_body,
        grid=(_NP // _BM,),
        in_specs=[pl.BlockSpec((_BM, _C), lambda i: (i, 0)),
                  pl.BlockSpec((_C, n_out), lambda i: (0, 0))],
        out_specs=pl.BlockSpec((_BM, n_out), lambda i: (i, 0)),
        out_shape=jax.ShapeDtypeStruct((_NP, n_out), jnp.float32),
    )(x, wcat)


def _bn_affine(p, bnw, bnb):
    psum = jnp.sum(p, axis=0)
    mu = psum[0:1] / float(_N)
    msq = psum[1:2] / float(_N)
    var = msq - mu * mu
    s = bnw * lax.rsqrt(var + 1e-5)
    t = bnb - mu * s
    return s, t


def _mm_fused_body(l_ref, w_ref, bnw_ref, bnb_ref, p_ref, y_ref):
    s, t = _bn_affine(p_ref[...], bnw_ref[...], bnb_ref[...])
    rows = pl.program_id(0) * _BM + lax.broadcasted_iota(jnp.int32, (_BM, 1), 0)
    a = jnp.where(rows < _N, l_ref[...] * s + t, 0.0)
    y_ref[...] = jnp.dot(a, w_ref[...], preferred_element_type=jnp.float32)


def _mm_fused(lmat, wcat, bnw, bnb, p):
    n_out = wcat.shape[1]
    return pl.pallas_call(
        _mm_fused_body,
        grid=(_NP // _BM,),
        in_specs=[pl.BlockSpec((_BM, _C), lambda i: (i, 0)),
                  pl.BlockSpec((_C, n_out), lambda i: (0, 0)),
                  pl.BlockSpec((1, _C), lambda i: (0, 0)),
                  pl.BlockSpec((1, _C), lambda i: (0, 0)),
                  pl.BlockSpec((_NW, 2, _C), lambda i: (0, 0, 0))],
        out_specs=pl.BlockSpec((_BM, n_out), lambda i: (i, 0)),
        out_shape=jax.ShapeDtypeStruct((_NP, n_out), jnp.float32),
    )(lmat, wcat, bnw, bnb, p)


def _final_body(l2_ref, l4_ref, p2_ref, p4_ref, w2_ref, b2_ref, w4_ref, b4_ref, o_ref):
    s2, t2 = _bn_affine(p2_ref[...], w2_ref[...], b2_ref[...])
    s4, t4 = _bn_affine(p4_ref[...], w4_ref[...], b4_ref[...])
    o_ref[...] = (l2_ref[...] * s2 + t2) + (l4_ref[...] * s4 + t4)


def _final(l2, p2, w2, b2, l4, p4, w4, b4):
    return pl.pallas_call(
        _final_body,
        grid=(_NP // _BM,),
        in_specs=[pl.BlockSpec((_BM, _C), lambda i: (i, 0)),
                  pl.BlockSpec((_BM, _C), lambda i: (i, 0)),
                  pl.BlockSpec((_NW, 2, _C), lambda i: (0, 0, 0)),
                  pl.BlockSpec((_NW, 2, _C), lambda i: (0, 0, 0)),
                  pl.BlockSpec((1, _C), lambda i: (0, 0)),
                  pl.BlockSpec((1, _C), lambda i: (0, 0)),
                  pl.BlockSpec((1, _C), lambda i: (0, 0)),
                  pl.BlockSpec((1, _C), lambda i: (0, 0))],
        out_specs=pl.BlockSpec((_BM, _C), lambda i: (i, 0)),
        out_shape=jax.ShapeDtypeStruct((_NP, _C), jnp.float32),
    )(l2, l4, p2, p4, w2, b2, w4, b4)


def _gather_body(tab_hbm, idx_hbm, l_hbm, p_hbm,
                 idx_v, rows_v, out_v, st_v, sg0, sg1, sw0, sw1):
    wid = lax.axis_index("s") * 2 + lax.axis_index("c")
    base = wid * _RPW
    zero16 = jnp.zeros((16,), jnp.float32)
    for a in range(2):
        for c in range(8):
            st_v[a, pl.ds(c * 16, 16)] = zero16

    sg = (sg0, sg1)
    sw = (sw0, sw1)

    pltpu.sync_copy(idx_hbm.at[wid], idx_v)

    def fire(cc, buf):
        for kk in range(_K):
            pltpu.async_copy(tab_hbm.at[idx_v.at[kk, pl.ds(cc * _B, _B)]],
                             rows_v.at[buf, kk], sg[buf])

    def drain_gathers(buf):
        for kk in range(_K):
            pltpu.make_async_copy(tab_hbm.at[idx_v.at[kk, pl.ds(0, _B)]],
                                  rows_v.at[buf, kk], sg[buf]).wait()

    def wait_write(buf):
        pltpu.make_async_copy(out_v.at[buf], l_hbm.at[pl.ds(0, _B)],
                              sw[buf]).wait()

    fire(0, 0)

    def pair(g, carry):
        for b in range(2):
            cc = 2 * g + b
            nxt = 1 - b
            if b == 0:
                fire(cc + 1, nxt)
            else:
                @pl.when(g < _CHUNKS // 2 - 1)
                def _():
                    fire(cc + 1, nxt)
            drain_gathers(b)

            @pl.when(g >= 1)
            def _():
                wait_write(b)

            def row(r, carry2):
                for c in range(8):
                    sl = pl.ds(c * 16, 16)
                    v = rows_v[b, 0, r, sl]
                    for kk in range(1, _K):
                        v = v + rows_v[b, kk, r, sl]
                    v = jnp.where(v >= 0.0, v, v * 0.01)
                    out_v[b, r, sl] = v
                    plsc.addupdate(st_v.at[0, sl], v)
                    plsc.addupdate(st_v.at[1, sl], v * v)
                return carry2

            lax.fori_loop(0, _B, row, 0)
            pltpu.async_copy(out_v.at[b], l_hbm.at[pl.ds(base + cc * _B, _B)],
                             sw[b])
        return carry

    lax.fori_loop(0, _CHUNKS // 2, pair, 0)
    wait_write(0)
    wait_write(1)
    pltpu.sync_copy(st_v, p_hbm.at[wid])


def _gather_sc(tab, idx):
    mesh = plsc.VectorSubcoreMesh(core_axis_name="c", subcore_axis_name="s")
    f = functools.partial(
        pl.kernel,
        out_type=(jax.ShapeDtypeStruct((_NP, _C), jnp.float32),
                  jax.ShapeDtypeStruct((_NW, 2, _C), jnp.float32)),
        mesh=mesh,
        scratch_types=[pltpu.VMEM((_K, _RPW), jnp.int32),
                       pltpu.VMEM((2, _K, _B, _C), jnp.float32),
                       pltpu.VMEM((2, _B, _C), jnp.float32),
                       pltpu.VMEM((2, _C), jnp.float32),
                       pltpu.SemaphoreType.DMA,
                       pltpu.SemaphoreType.DMA,
                       pltpu.SemaphoreType.DMA,
                       pltpu.SemaphoreType.DMA],
    )(_gather_body)
    return f(tab, idx)


def kernel(features, coords, W1, W1_2, W2, W3,
           bn0_w, bn0_b, bn0_2_w, bn0_2_b, bn1_w, bn1_b, bn2_w, bn2_b):
    f32 = jnp.float32
    x = jnp.zeros((_NP, _C), f32).at[:_N].set(features)
    grid = jnp.full(_GRID, -1, jnp.int32).at[
        coords[:, 0], coords[:, 1], coords[:, 2]].set(jnp.arange(_N, dtype=jnp.int32))
    idx_a = _nbr_idx(grid, coords, _OFF_A)
    idx_b = _nbr_idx(grid, coords, _OFF_B)

    w1c, w12c, w2c, w3c = _cat9(W1), _cat9(W1_2), _cat9(W2), _cat9(W3)
    bn = lambda v: v.reshape(1, _C)

    y1 = _mm_plain(x, w1c).reshape(_NP * _K, _C)
    l1, p1 = _gather_sc(y1, idx_a)
    y2 = _mm_fused(l1, w12c, bn(bn0_w), bn(bn0_b), p1).reshape(_NP * _K, _C)
    l2, p2 = _gather_sc(y2, idx_b)

    y3 = _mm_plain(x, w2c).reshape(_NP * _K, _C)
    l3, p3 = _gather_sc(y3, idx_b)
    y4 = _mm_fused(l3, w3c, bn(bn1_w), bn(bn1_b), p3).reshape(_NP * _K, _C)
    l4, p4 = _gather_sc(y4, idx_a)

    out = _final(l2, p2, bn(bn0_2_w), bn(bn0_2_b), l4, p4, bn(bn2_w), bn(bn2_b))
    return out[:_N]
